# baseline (speedup 1.0000x reference)
import functools

import jax
import jax.numpy as jnp
from jax import lax
from jax.experimental import pallas as pl
from jax.experimental.pallas import tpu as pltpu
from jax.experimental.pallas import tpu_sc as plsc

NC = 2
NS = 16
NW = NC * NS

NB = 128
NSLOT = 8
LAG = 4


@functools.partial(jax.jit, static_argnames=("B", "F", "V", "H"))
def _embed_gather(Xt, tables, *, B, F, V, H):
    spw = B // NW
    n_chunks = spw // NB
    assert spw % NB == 0
    mesh = plsc.VectorSubcoreMesh(
        core_axis_name="c", subcore_axis_name="s",
        num_cores=NC, num_subcores=NS)

    @functools.partial(
        pl.kernel,
        out_type=jax.ShapeDtypeStruct((B, F, H), jnp.float32),
        mesh=mesh,
        scratch_types=[
            pltpu.VMEM((2, F, NB), jnp.int32),
            pltpu.VMEM((NSLOT, NB, H), jnp.float32),
            [pltpu.SemaphoreType.DMA] * NSLOT,
            [pltpu.SemaphoreType.DMA] * NSLOT,
        ],
        compiler_params=pltpu.CompilerParams(use_tc_tiling_on_sc=False),
    )
    def k(xt_hbm, tab_hbm, out_hbm, xtbuf, rows, gsems, osems):
        wid = lax.axis_index("s") * NC + lax.axis_index("c")
        base = wid * spw

        steps = [(c, f) for c in range(n_chunks) for f in range(F)]

        def gather_of(i):
            c, f = steps[i]
            slot = i % NSLOT
            return pltpu.make_async_copy(
                tab_hbm.at[f].at[xtbuf.at[c % 2, f]], rows.at[slot],
                gsems[slot])

        def write_of(i):
            c, f = steps[i]
            slot = i % NSLOT
            b0 = base + c * NB
            return pltpu.make_async_copy(
                rows.at[slot], out_hbm.at[pl.ds(b0, NB), f, :], osems[slot])

        for i, (c, f) in enumerate(steps):
            if f == 0:
                pltpu.sync_copy(
                    xt_hbm.at[:, pl.ds(base + c * NB, NB)],
                    xtbuf.at[c % 2])
            if i >= LAG:
                gather_of(i - LAG).wait()
                write_of(i - LAG).start()
            if i >= NSLOT:
                write_of(i - NSLOT).wait()
            gather_of(i).start()
        n = len(steps)
        for i in range(max(0, n - LAG), n):
            gather_of(i).wait()
            write_of(i).start()
        for i in range(max(0, n - NSLOT), n):
            write_of(i).wait()

    return k(Xt, tables)


def kernel(X, tables):
    F, V, H = tables.shape
    B = X.shape[0]
    Xt = X.T.astype(jnp.int32)
    out = _embed_gather(Xt, tables, B=B, F=F, V=V, H=H)
    return out.reshape(B, F * H)


# speedup vs baseline: 1.5572x; 1.5572x over previous
import functools

import jax
import jax.numpy as jnp
from jax import lax
from jax.experimental import pallas as pl
from jax.experimental.pallas import tpu as pltpu
from jax.experimental.pallas import tpu_sc as plsc

NC = 2
NS = 16
NW = NC * NS

SAMP = 8
LANES = 16


@functools.partial(jax.jit, static_argnames=("B", "F", "V", "H"))
def _embed_gather(X_flat, tables, *, B, F, V, H):
    spw = B // NW
    n_blocks = spw // SAMP
    rows_pb = SAMP * F
    assert spw % SAMP == 0 and n_blocks % 2 == 0
    mesh = plsc.VectorSubcoreMesh(
        core_axis_name="c", subcore_axis_name="s",
        num_cores=NC, num_subcores=NS)

    @functools.partial(
        pl.kernel,
        out_type=jax.ShapeDtypeStruct((B, F * H), jnp.float32),
        mesh=mesh,
        scratch_types=[
            pltpu.VMEM((spw * F,), jnp.int32),
            pltpu.SemaphoreType.DMA,
            pltpu.VMEM((2, rows_pb, H), jnp.float32),
            pltpu.VMEM((2, SAMP, F * H), jnp.float32),
            [pltpu.SemaphoreType.DMA] * 2,
            [pltpu.SemaphoreType.DMA] * 2,
        ],
    )
    def k(x_hbm, tab_hbm, out_hbm, xvm, xsem, rows, obuf, gsems, osems):
        wid = lax.axis_index("s") * NC + lax.axis_index("c")
        sbase = wid * spw
        pltpu.async_copy(
            x_hbm.at[pl.ds(sbase * F, spw * F)], xvm, xsem).wait()

        def block(n, s):
            b0 = sbase + n * SAMP

            @pl.when(n >= 2)
            def _drain_prev_write():
                pltpu.make_async_copy(
                    obuf.at[s], out_hbm.at[pl.ds(b0 - 2 * SAMP, SAMP)],
                    osems[s]).wait()

            def fire_rows(i, carry):
                j0 = i * F
                va = xvm[pl.ds(n * rows_pb + j0, LANES)]
                vb = xvm[pl.ds(n * rows_pb + j0 + LANES, LANES)]
                for f in range(F):
                    row = va[f] if f < LANES else vb[f - LANES]
                    pltpu.async_copy(
                        tab_hbm.at[f, pl.ds(row, 1), :],
                        rows.at[s, pl.ds(j0 + f, 1), :],
                        gsems[s])
                return carry

            lax.fori_loop(0, SAMP, fire_rows, 0)
            pltpu.make_async_copy(
                tab_hbm.at[0, pl.ds(0, rows_pb), :], rows.at[s],
                gsems[s]).wait()

            def assemble(i, carry):
                j0 = i * F
                for f in range(F):
                    for t in range(H // LANES):
                        obuf[s, i, pl.ds(f * H + t * LANES, LANES)] = (
                            rows[s, j0 + f, pl.ds(t * LANES, LANES)])
                return carry

            lax.fori_loop(0, SAMP, assemble, 0)
            pltpu.async_copy(
                obuf.at[s], out_hbm.at[pl.ds(b0, SAMP)], osems[s])

        def pair(p, carry):
            block(2 * p, 0)
            block(2 * p + 1, 1)
            return carry

        lax.fori_loop(0, n_blocks // 2, pair, 0)
        for s in range(2):
            pltpu.make_async_copy(
                obuf.at[s],
                out_hbm.at[pl.ds(sbase + (n_blocks - 2 + s) * SAMP, SAMP)],
                osems[s]).wait()

    return k(X_flat, tables)


def kernel(X, tables):
    F, V, H = tables.shape
    B = X.shape[0]
    X_flat = X.reshape(B * F).astype(jnp.int32)
    return _embed_gather(X_flat, tables, B=B, F=F, V=V, H=H)


# speedup vs baseline: 1.5670x; 1.0063x over previous
import functools

import jax
import jax.numpy as jnp
from jax import lax
from jax.experimental import pallas as pl
from jax.experimental.pallas import tpu as pltpu
from jax.experimental.pallas import tpu_sc as plsc

NC = 2
NS = 16
NW = NC * NS

SAMP = 8
LANES = 16
XPAD = 128


@functools.partial(jax.jit, static_argnames=("B", "F", "V", "H"))
def _embed_gather(Xp, tables, *, B, F, V, H):
    spw = B // NW
    n_blocks = spw // SAMP
    rows_pb = SAMP * F
    assert spw % SAMP == 0 and n_blocks % 2 == 0
    mesh = plsc.VectorSubcoreMesh(
        core_axis_name="c", subcore_axis_name="s",
        num_cores=NC, num_subcores=NS)

    @functools.partial(
        pl.kernel,
        out_type=jax.ShapeDtypeStruct((B, F * H), jnp.float32),
        mesh=mesh,
        scratch_types=[
            pltpu.VMEM((2, SAMP, XPAD), jnp.int32),
            pltpu.VMEM((2, rows_pb, H), jnp.float32),
            pltpu.VMEM((2, SAMP, F * H), jnp.float32),
            [pltpu.SemaphoreType.DMA] * 2,
            [pltpu.SemaphoreType.DMA] * 2,
            [pltpu.SemaphoreType.DMA] * 2,
        ],
    )
    def k(x_hbm, tab_hbm, out_hbm, xbuf, rows, obuf, xsems, gsems, osems):
        wid = lax.axis_index("s") * NC + lax.axis_index("c")
        sbase = wid * spw

        for s in range(2):
            pltpu.async_copy(
                x_hbm.at[pl.ds(sbase + s * SAMP, SAMP)], xbuf.at[s],
                xsems[s])

        def block(n, s):
            b0 = sbase + n * SAMP

            @pl.when(n >= 2)
            def _drain_prev_write():
                pltpu.make_async_copy(
                    obuf.at[s], out_hbm.at[pl.ds(b0 - 2 * SAMP, SAMP)],
                    osems[s]).wait()

            pltpu.make_async_copy(
                x_hbm.at[pl.ds(b0, SAMP)], xbuf.at[s], xsems[s]).wait()

            def fire_rows(i, carry):
                j0 = i * F
                va = xbuf[s, i, pl.ds(0, LANES)]
                vb = xbuf[s, i, pl.ds(LANES, LANES)]
                for f in range(F):
                    row = va[f] if f < LANES else vb[f - LANES]
                    pltpu.async_copy(
                        tab_hbm.at[f, pl.ds(row, 1), :],
                        rows.at[s, pl.ds(j0 + f, 1), :],
                        gsems[s])
                return carry

            lax.fori_loop(0, SAMP, fire_rows, 0)

            @pl.when(n + 2 < n_blocks)
            def _prefetch_x():
                pltpu.async_copy(
                    x_hbm.at[pl.ds(b0 + 2 * SAMP, SAMP)], xbuf.at[s],
                    xsems[s])

            pltpu.make_async_copy(
                tab_hbm.at[0, pl.ds(0, rows_pb), :], rows.at[s],
                gsems[s]).wait()

            def assemble(i, carry):
                j0 = i * F
                for f in range(F):
                    for t in range(H // LANES):
                        obuf[s, i, pl.ds(f * H + t * LANES, LANES)] = (
                            rows[s, j0 + f, pl.ds(t * LANES, LANES)])
                return carry

            lax.fori_loop(0, SAMP, assemble, 0)
            pltpu.async_copy(
                obuf.at[s], out_hbm.at[pl.ds(b0, SAMP)], osems[s])

        def pair(p, carry):
            block(2 * p, 0)
            block(2 * p + 1, 1)
            return carry

        lax.fori_loop(0, n_blocks // 2, pair, 0)
        for s in range(2):
            pltpu.make_async_copy(
                obuf.at[s],
                out_hbm.at[pl.ds(sbase + (n_blocks - 2 + s) * SAMP, SAMP)],
                osems[s]).wait()

    return k(Xp, tables)


def kernel(X, tables):
    F, V, H = tables.shape
    B = X.shape[0]
    Xp = jnp.pad(X.astype(jnp.int32), ((0, 0), (0, XPAD - F)))
    return _embed_gather(Xp, tables, B=B, F=F, V=V, H=H)
